# Initial kernel scaffold; baseline (speedup 1.0000x reference)
import jax
import jax.numpy as jnp
from jax.experimental import pallas as pl


def kernel(queue, inp, vid_idx):
    raise NotImplementedError("write your pallas kernel here")


# speedup vs baseline: 3.0060x; 3.0060x over previous
import functools

import jax
import jax.numpy as jnp
from jax import lax
from jax.experimental import pallas as pl
from jax.experimental.pallas import tpu as pltpu
from jax.experimental.pallas import tpu_sc as plsc

_N_VIDEO = 100000
_N_MU = 8
_OUT_DIM = 64
_BATCH = 16384
_ROW = _N_MU * _OUT_DIM
_MOM = 0.9

_NC = 2
_NS = 16
_NW = _NC * _NS
_B_PER_W = _BATCH // _NW
_CHUNK = 64
_NCHUNK = _B_PER_W // _CHUNK
_LANE = 16
_VECS = _CHUNK * _ROW // _LANE


def _update_body(q_hbm, i_hbm, vid_hbm, bwin_hbm, out_ref,
                 idx_v, bwin_v, qbuf, ibuf, gsem, isem, ssem):
    w = lax.axis_index("s") * _NC + lax.axis_index("c")
    pltpu.sync_copy(vid_hbm.at[w], idx_v)
    pltpu.sync_copy(bwin_hbm.at[w], bwin_v)

    for j in range(_NCHUNK):
        cq = pltpu.async_copy(q_hbm.at[idx_v.at[j]], qbuf, gsem)
        ci = pltpu.async_copy(i_hbm.at[bwin_v.at[j]], ibuf, isem)
        cq.wait()
        ci.wait()

        @pl.loop(0, _VECS)
        def _blend(i):
            r = i // (_ROW // _LANE)
            c = (i % (_ROW // _LANE)) * _LANE
            q = qbuf[r, pl.ds(c, _LANE)]
            x = ibuf[r, pl.ds(c, _LANE)]
            qbuf[r, pl.ds(c, _LANE)] = q * _MOM + x * (1.0 - _MOM)

        pltpu.async_copy(qbuf, out_ref.at[idx_v.at[j]], ssem).wait()


@functools.cache
def _get_update():
    mesh = plsc.VectorSubcoreMesh(
        core_axis_name="c", subcore_axis_name="s", num_cores=_NC,
        num_subcores=_NS)
    return pl.kernel(
        _update_body,
        out_type=(),
        mesh=mesh,
        scratch_types=[
            pltpu.VMEM((_NCHUNK, _CHUNK), jnp.int32),
            pltpu.VMEM((_NCHUNK, _CHUNK), jnp.int32),
            pltpu.VMEM((_CHUNK, _ROW), jnp.float32),
            pltpu.VMEM((_CHUNK, _ROW), jnp.float32),
            pltpu.SemaphoreType.DMA,
            pltpu.SemaphoreType.DMA,
            pltpu.SemaphoreType.DMA,
        ],
    )


@jax.jit
def kernel(queue, inp, vid_idx):
    qflat = queue.reshape(_N_VIDEO, _ROW)
    iflat = inp.reshape(_BATCH, _ROW)
    b_idx = jnp.arange(_BATCH, dtype=jnp.int32)
    wtab = jnp.zeros((_N_VIDEO,), jnp.int32).at[vid_idx].max(b_idx)
    b_win = wtab[vid_idx]
    vid3 = vid_idx.reshape(_NW, _NCHUNK, _CHUNK)
    bwin3 = b_win.reshape(_NW, _NCHUNK, _CHUNK)
    out_ref = jax.new_ref(qflat)
    _get_update()(qflat, iflat, vid3, bwin3, out_ref)
    return out_ref[...].reshape(_N_VIDEO, _N_MU, _OUT_DIM)
